# baseline (speedup 1.0000x reference)
import functools

import jax
import jax.numpy as jnp
from jax import lax
from jax.experimental import pallas as pl
from jax.experimental.pallas import tpu as pltpu

_BLOCK_N = 1048576


def _linear_t_kernel(x_ref, w_ref, b_ref, o_ref):
    acc = jnp.dot(w_ref[...], x_ref[...],
                  preferred_element_type=jnp.float32)
    o_ref[...] = (acc + b_ref[...]).astype(o_ref.dtype)


@functools.partial(jax.jit, static_argnames=("block_n",))
def _linear_t(x_t, weight, bias2d, *, block_n):
    in_f, cols = x_t.shape
    out_f = weight.shape[0]
    grid = (pl.cdiv(cols, block_n),)
    return pl.pallas_call(
        _linear_t_kernel,
        out_shape=jax.ShapeDtypeStruct((out_f, cols), x_t.dtype),
        grid=grid,
        in_specs=[
            pl.BlockSpec((in_f, block_n), lambda i: (0, i)),
            pl.BlockSpec((out_f, in_f), lambda i: (0, 0)),
            pl.BlockSpec((out_f, 1), lambda i: (0, 0)),
        ],
        out_specs=pl.BlockSpec((out_f, block_n), lambda i: (0, i)),
        compiler_params=pltpu.CompilerParams(
            dimension_semantics=("arbitrary",)),
    )(x_t, weight, bias2d)


def kernel(x, weight, bias):
    x_t = x.T
    y_t = _linear_t(x_t, weight, bias.reshape(-1, 1), block_n=_BLOCK_N)
    return y_t.T


# speedup vs baseline: 1.0074x; 1.0074x over previous
import functools

import jax
import jax.numpy as jnp
from jax import lax
from jax.experimental import pallas as pl
from jax.experimental.pallas import tpu as pltpu

_BLOCK_N = 524288


def _linear_t_kernel(x_ref, w_ref, b_ref, o_ref):
    acc = jnp.dot(w_ref[...], x_ref[...],
                  preferred_element_type=jnp.float32)
    o_ref[...] = (acc + b_ref[...]).astype(o_ref.dtype)


@functools.partial(jax.jit, static_argnames=("block_n",))
def _linear_t(x_t, weight, bias2d, *, block_n):
    in_f, cols = x_t.shape
    out_f = weight.shape[0]
    grid = (pl.cdiv(cols, block_n),)
    return pl.pallas_call(
        _linear_t_kernel,
        out_shape=jax.ShapeDtypeStruct((out_f, cols), x_t.dtype),
        grid=grid,
        in_specs=[
            pl.BlockSpec((in_f, block_n), lambda i: (0, i)),
            pl.BlockSpec((out_f, in_f), lambda i: (0, 0)),
            pl.BlockSpec((out_f, 1), lambda i: (0, 0)),
        ],
        out_specs=pl.BlockSpec((out_f, block_n), lambda i: (0, i)),
        compiler_params=pltpu.CompilerParams(
            dimension_semantics=("arbitrary",)),
    )(x_t, weight, bias2d)


def kernel(x, weight, bias):
    x_t = x.T
    y_t = _linear_t(x_t, weight, bias.reshape(-1, 1), block_n=_BLOCK_N)
    return y_t.T


# speedup vs baseline: 1.0168x; 1.0094x over previous
import functools

import jax
import jax.numpy as jnp
from jax.experimental import pallas as pl
from jax.experimental.pallas import tpu as pltpu

_BLOCK_N = 524288


def _linear_t_kernel(x_ref, w_ref, b_ref, o_ref):
    acc = jnp.dot(w_ref[...], x_ref[...],
                  preferred_element_type=jnp.float32)
    o_ref[...] = (acc + b_ref[...]).astype(o_ref.dtype)


@functools.partial(jax.jit, static_argnames=("block_n",))
def _linear_t(x_t, weight, bias2d, *, block_n):
    in_f, cols = x_t.shape
    out_f = weight.shape[0]
    grid = (pl.cdiv(cols, block_n),)
    return pl.pallas_call(
        _linear_t_kernel,
        out_shape=jax.ShapeDtypeStruct((out_f, cols), x_t.dtype),
        grid=grid,
        in_specs=[
            pl.BlockSpec((in_f, block_n), lambda i: (0, i)),
            pl.BlockSpec((out_f, in_f), lambda i: (0, 0)),
            pl.BlockSpec((out_f, 1), lambda i: (0, 0)),
        ],
        out_specs=pl.BlockSpec((out_f, block_n), lambda i: (0, i)),
        compiler_params=pltpu.CompilerParams(
            dimension_semantics=("parallel",)),
    )(x_t, weight, bias2d)


def kernel(x, weight, bias):
    x_t = x.T
    y_t = _linear_t(x_t, weight, bias.reshape(-1, 1), block_n=_BLOCK_N)
    return y_t.T
